# Initial kernel scaffold; baseline (speedup 1.0000x reference)
import jax
import jax.numpy as jnp
from jax.experimental import pallas as pl


def kernel(embedding_sequence, pos_table):
    raise NotImplementedError("write your pallas kernel here")


# speedup vs baseline: 1.6690x; 1.6690x over previous
import jax
import jax.numpy as jnp
from jax.experimental import pallas as pl
from jax.experimental.pallas import tpu as pltpu

_BS = 1024


def _add_kernel(emb_ref, tab_ref, out_ref):
    out_ref[...] = emb_ref[...] + tab_ref[...][None, :, :]


def kernel(embedding_sequence, pos_table):
    batch, seq_len, embed = embedding_sequence.shape
    n_s = seq_len // _BS
    grid = (n_s, batch)
    return pl.pallas_call(
        _add_kernel,
        grid=grid,
        in_specs=[
            pl.BlockSpec((1, _BS, embed), lambda s, b: (b, s, 0)),
            pl.BlockSpec((_BS, embed), lambda s, b: (s, 0)),
        ],
        out_specs=pl.BlockSpec((1, _BS, embed), lambda s, b: (b, s, 0)),
        out_shape=jax.ShapeDtypeStruct((batch, seq_len, embed), embedding_sequence.dtype),
        compiler_params=pltpu.CompilerParams(
            dimension_semantics=("arbitrary", "arbitrary"),
        ),
    )(embedding_sequence, pos_table)
